# Initial kernel scaffold; baseline (speedup 1.0000x reference)
import functools

import jax
import jax.numpy as jnp
from jax import lax
from jax.experimental import pallas as pl
from jax.experimental.pallas import tpu as pltpu
from jax.experimental.pallas import tpu_sc as plsc

_N = 10000
_D = 128
_E = 320000

_NC = 2
_NS = 16
_NW = _NC * _NS

_CHUNK = 512
_SUB = _CHUNK // 128
_CH_PER_W = 20
_EPW = _CHUNK * _CH_PER_W
_E_PAD = _EPW * _NW
_ROWS_PER_W = _EPW // 128
_RPS = _N // _NS


def _mm_body(x_ref, w_ref, o_ref):
    o_ref[...] = jnp.dot(x_ref[...], w_ref[...],
                         preferred_element_type=jnp.float32)


def _matmul(x, W):
    return pl.pallas_call(
        _mm_body,
        grid=(10,),
        in_specs=[
            pl.BlockSpec((_N // 10, _D), lambda i: (i, 0)),
            pl.BlockSpec((_D, _D), lambda i: (0, 0)),
        ],
        out_specs=pl.BlockSpec((_N // 10, _D), lambda i: (i, 0)),
        out_shape=jax.ShapeDtypeStruct((_N, _D), jnp.float32),
    )(x, W)


def _combine_body(p_ref, o_ref):
    o_ref[...] = p_ref[0] + p_ref[1]


def _combine(partials):
    return pl.pallas_call(
        _combine_body,
        grid=(10,),
        in_specs=[pl.BlockSpec((2, _N // 10, _D), lambda i: (0, i, 0))],
        out_specs=pl.BlockSpec((_N // 10, _D), lambda i: (i, 0)),
        out_shape=jax.ShapeDtypeStruct((_N, _D), jnp.float32),
    )(partials)


_mesh = plsc.VectorSubcoreMesh(core_axis_name="c", subcore_axis_name="s")


@functools.partial(
    pl.kernel,
    mesh=_mesh,
    out_type=jax.ShapeDtypeStruct((_NC, _N, _D), jnp.float32),
    scratch_types=[
        pltpu.VMEM((_SUB, 128), jnp.int32),
        pltpu.VMEM((_SUB, 128), jnp.int32),
        pltpu.VMEM((_SUB, 128), jnp.float32),
        pltpu.VMEM((_CHUNK, _D), jnp.float32),
        pltpu.VMEM_SHARED((_N, _D), jnp.float32),
        pltpu.SemaphoreType.DMA,
    ],
)
def _sc_spmm(support_hbm, src_hbm, dst_hbm, vals_hbm, out_hbm,
             srcv, dstv, valsv, rows, acc, sem):
    c = lax.axis_index("c")
    s = lax.axis_index("s")
    wid = c * _NS + s

    zv = jnp.zeros((16,), jnp.float32)

    def _zrow(i, carry):
        for q in range(_D // 16):
            rows[i, pl.ds(q * 16, 16)] = zv
        return carry

    lax.fori_loop(0, _CHUNK, _zrow, 0)
    r0 = s * _RPS
    pltpu.sync_copy(rows.at[pl.ds(0, _CHUNK)], acc.at[pl.ds(r0, _CHUNK)])
    pltpu.sync_copy(rows.at[pl.ds(0, _RPS - _CHUNK)],
                    acc.at[pl.ds(r0 + _CHUNK, _RPS - _CHUNK)])
    plsc.subcore_barrier()

    def _chunk_body(i, carry):
        row0 = wid * _ROWS_PER_W + i * _SUB
        pltpu.sync_copy(src_hbm.at[pl.ds(row0, _SUB)], srcv)
        pltpu.sync_copy(dst_hbm.at[pl.ds(row0, _SUB)], dstv)
        pltpu.sync_copy(vals_hbm.at[pl.ds(row0, _SUB)], valsv)
        cps = [
            pltpu.async_copy(support_hbm.at[srcv.at[j]],
                             rows.at[pl.ds(j * 128, 128)], sem)
            for j in range(_SUB)
        ]
        for cp in cps:
            cp.wait()
        for j in range(_SUB):
            def _scale_one(e, cc, j=j):
                val = valsv[j, e]
                r = j * 128 + e
                for q in range(_D // 16):
                    sl = pl.ds(q * 16, 16)
                    rows[r, sl] = rows[r, sl] * val
                return cc
            lax.fori_loop(0, 128, _scale_one, 0)
        for j in range(_SUB):
            pltpu.sync_copy(rows.at[pl.ds(j * 128, 128)],
                            acc.at[dstv.at[j]], add=True)
        return carry

    lax.fori_loop(0, _CH_PER_W, _chunk_body, 0)

    plsc.subcore_barrier()
    out_sc = out_hbm.at[c]
    pltpu.sync_copy(acc.at[pl.ds(r0, _RPS)], out_sc.at[pl.ds(r0, _RPS)])


def kernel(x, W, adj_vals, src, dst):
    support = _matmul(x, W)
    pad = _E_PAD - _E
    src_p = jnp.concatenate([src, jnp.zeros((pad,), jnp.int32)])
    dst_p = jnp.concatenate([dst, jnp.zeros((pad,), jnp.int32)])
    vals_p = jnp.concatenate([adj_vals, jnp.zeros((pad,), jnp.float32)])
    src2 = src_p.reshape(_E_PAD // 128, 128)
    dst2 = dst_p.reshape(_E_PAD // 128, 128)
    vals2 = vals_p.reshape(_E_PAD // 128, 128)
    partials = _sc_spmm(support, src2, dst2, vals2)
    return _combine(partials)


# speedup vs baseline: 3.8374x; 3.8374x over previous
import functools

import jax
import jax.numpy as jnp
from jax import lax
from jax.experimental import pallas as pl
from jax.experimental.pallas import tpu as pltpu
from jax.experimental.pallas import tpu_sc as plsc

_N = 10000
_D = 128
_E = 320000

_NC = 2
_NS = 16
_NW = _NC * _NS

_G = 128
_CPG = 8
_NCHUNK = 10
_GROUPS = _CPG * _NCHUNK
_EPW = _G * _GROUPS
_E_PAD = _EPW * _NW
_ROWS_PER_W = _EPW // 128
_RPS = 624


def _mm_body(x_ref, w_ref, o_ref):
    o_ref[...] = jnp.dot(x_ref[...], w_ref[...],
                         preferred_element_type=jnp.float32)


def _matmul(x, W):
    return pl.pallas_call(
        _mm_body,
        grid=(10,),
        in_specs=[
            pl.BlockSpec((_N // 10, _D), lambda i: (i, 0)),
            pl.BlockSpec((_D, _D), lambda i: (0, 0)),
        ],
        out_specs=pl.BlockSpec((_N // 10, _D), lambda i: (i, 0)),
        out_shape=jax.ShapeDtypeStruct((_N, _D), jnp.float32),
    )(x, W)


def _combine_body(p_ref, o_ref):
    o_ref[...] = p_ref[0] + p_ref[1]


def _combine(partials):
    return pl.pallas_call(
        _combine_body,
        grid=(10,),
        in_specs=[pl.BlockSpec((2, _N // 10, _D), lambda i: (0, i, 0))],
        out_specs=pl.BlockSpec((_N // 10, _D), lambda i: (i, 0)),
        out_shape=jax.ShapeDtypeStruct((_N, _D), jnp.float32),
    )(partials)


_mesh = plsc.VectorSubcoreMesh(core_axis_name="c", subcore_axis_name="s")


@functools.partial(
    pl.kernel,
    mesh=_mesh,
    out_type=jax.ShapeDtypeStruct((_NC, _N, _D), jnp.float32),
    scratch_types=[
        pltpu.VMEM((2 * _CPG, 128), jnp.int32),
        pltpu.VMEM((2 * _CPG, 128), jnp.int32),
        pltpu.VMEM((2 * _CPG, 128), jnp.float32),
        pltpu.VMEM((2 * _G, _D), jnp.float32),
        pltpu.VMEM_SHARED((_N, _D), jnp.float32),
        pltpu.SemaphoreType.DMA,
        pltpu.SemaphoreType.DMA,
        pltpu.SemaphoreType.DMA,
    ],
)
def _sc_spmm(support_hbm, src_hbm, dst_hbm, vals_hbm, out_hbm,
             srcv, dstv, valsv, rows, acc, sem_g, sem_s, sem_i):
    c = lax.axis_index("c")
    s = lax.axis_index("s")
    wid = c * _NS + s

    zv = jnp.zeros((16,), jnp.float32)

    def _zrow(i, carry):
        for q in range(_D // 16):
            rows[i, pl.ds(q * 16, 16)] = zv
        return carry

    lax.fori_loop(0, 2 * _G, _zrow, 0)
    r0 = s * _RPS
    pltpu.sync_copy(rows.at[pl.ds(0, 2 * _G)], acc.at[pl.ds(r0, 2 * _G)])
    pltpu.sync_copy(rows.at[pl.ds(0, 2 * _G)],
                    acc.at[pl.ds(r0 + 2 * _G, 2 * _G)])
    pltpu.sync_copy(rows.at[pl.ds(0, _RPS - 4 * _G)],
                    acc.at[pl.ds(r0 + 4 * _G, _RPS - 4 * _G)])

    @pl.when(s == _NS - 1)
    def _zero_tail():
        pltpu.sync_copy(rows.at[pl.ds(0, _N - _NS * _RPS)],
                        acc.at[pl.ds(_NS * _RPS, _N - _NS * _RPS)])

    plsc.subcore_barrier()

    hbm_idx = (src_hbm, dst_hbm, vals_hbm)
    vmem_idx = (srcv, dstv, valsv)
    base_row = wid * _ROWS_PER_W

    def _fire_idx(chunk, buf):
        for hb, vb in zip(hbm_idx, vmem_idx):
            pltpu.async_copy(hb.at[pl.ds(base_row + chunk * _CPG, _CPG)],
                             vb.at[pl.ds(buf * _CPG, _CPG)], sem_i)

    def _wait_idx():
        for hb, vb in zip(hbm_idx, vmem_idx):
            pltpu.make_async_copy(hb.at[pl.ds(0, _CPG)],
                                  vb.at[pl.ds(0, _CPG)], sem_i).wait()

    def _fire_gather(idx_row, dst_half):
        pltpu.async_copy(support_hbm.at[srcv.at[idx_row]],
                         rows.at[pl.ds(dst_half * _G, _G)], sem_g)

    def _wait_gather():
        pltpu.make_async_copy(support_hbm.at[pl.ds(0, _G)],
                              rows.at[pl.ds(0, _G)], sem_g).wait()

    def _fire_scatter(src_half, idx_row):
        pltpu.async_copy(rows.at[pl.ds(src_half * _G, _G)],
                         acc.at[dstv.at[idx_row]], sem_s, add=True)

    def _wait_scatter():
        pltpu.make_async_copy(support_hbm.at[pl.ds(0, _G)],
                              rows.at[pl.ds(0, _G)], sem_s).wait()

    def _scale(half, vrow):
        def _grp(g16, cc):
            vv = valsv[vrow, pl.ds(g16 * 16, 16)]
            base = half * _G + g16 * 16
            for t in range(16):
                val = vv[t]
                r = base + t
                for q in range(_D // 16):
                    sl = pl.ds(q * 16, 16)
                    rows[r, sl] = rows[r, sl] * val
            return cc
        lax.fori_loop(0, _G // 16, _grp, 0)

    for hb, vb in zip(hbm_idx, vmem_idx):
        pltpu.sync_copy(hb.at[pl.ds(base_row, _CPG)], vb.at[pl.ds(0, _CPG)])
    _fire_gather(0, 0)

    def _chunk_body(ch, carry):
        cb = ch % 2
        for j in range(_CPG):
            cur = j % 2
            nxt = (j + 1) % 2
            if j > 0:
                _wait_scatter()
            else:
                @pl.when(ch > 0)
                def _w():
                    _wait_scatter()
                @pl.when(ch < _NCHUNK - 1)
                def _p():
                    _fire_idx(ch + 1, 1 - cb)
            if j < _CPG - 1:
                _fire_gather(cb * _CPG + j + 1, nxt)
            else:
                @pl.when(ch < _NCHUNK - 1)
                def _g():
                    _wait_idx()
                    _fire_gather((1 - cb) * _CPG, nxt)
            _wait_gather()
            _scale(cur, cb * _CPG + j)
            _fire_scatter(cur, cb * _CPG + j)
        return carry

    lax.fori_loop(0, _NCHUNK, _chunk_body, 0)
    _wait_scatter()

    plsc.subcore_barrier()
    out_sc = out_hbm.at[c]
    pltpu.sync_copy(acc.at[pl.ds(r0, _RPS)], out_sc.at[pl.ds(r0, _RPS)])

    @pl.when(s == _NS - 1)
    def _write_tail():
        pltpu.sync_copy(acc.at[pl.ds(_NS * _RPS, _N - _NS * _RPS)],
                        out_sc.at[pl.ds(_NS * _RPS, _N - _NS * _RPS)])


def kernel(x, W, adj_vals, src, dst):
    support = _matmul(x, W)
    pad = _E_PAD - _E
    src_p = jnp.concatenate([src, jnp.zeros((pad,), jnp.int32)])
    dst_p = jnp.concatenate([dst, jnp.zeros((pad,), jnp.int32)])
    vals_p = jnp.concatenate([adj_vals, jnp.zeros((pad,), jnp.float32)])
    src2 = src_p.reshape(_E_PAD // 128, 128)
    dst2 = dst_p.reshape(_E_PAD // 128, 128)
    vals2 = vals_p.reshape(_E_PAD // 128, 128)
    partials = _sc_spmm(support, src2, dst2, vals2)
    return _combine(partials)
